# baseline (speedup 1.0000x reference)
import functools

import jax
import jax.numpy as jnp
from jax import lax
from jax.experimental import pallas as pl
from jax.experimental.pallas import tpu as pltpu
from jax.experimental.pallas import tpu_sc as plsc

D = 128
NW = 32
CHUNK = 128
NBUF = 5
LAG = 3


def _make_gather(bat, hist):
    nch = hist
    mesh = plsc.VectorSubcoreMesh(core_axis_name="c", subcore_axis_name="s")

    @functools.partial(
        pl.kernel,
        out_type=jax.ShapeDtypeStruct((hist, bat, D), jnp.float32),
        mesh=mesh,
        scratch_types=[
            pltpu.VMEM((nch, CHUNK), jnp.int32),
            pltpu.VMEM((NBUF, CHUNK, D), jnp.float32),
        ] + [pltpu.SemaphoreType.DMA] * (2 * NBUF),
    )
    def gather(table_hbm, idx_hbm, out_hbm, idx_v, bufs, *sems):
        gsem = sems[:NBUF]
        wsem = sems[NBUF:]
        wid = lax.axis_index("s") * 2 + lax.axis_index("c")
        col = wid * CHUNK
        pltpu.sync_copy(idx_hbm.at[wid], idx_v)

        def issue_gather(j, b):
            pltpu.async_copy(table_hbm.at[idx_v.at[j]], bufs.at[b], gsem[b])

        def wait_gather(j, b):
            pltpu.make_async_copy(
                table_hbm.at[idx_v.at[j]], bufs.at[b], gsem[b]
            ).wait()

        def issue_write(j, b):
            pltpu.async_copy(
                bufs.at[b], out_hbm.at[j, pl.ds(col, CHUNK)], wsem[b]
            )

        def wait_write(j, b):
            pltpu.make_async_copy(
                bufs.at[b], out_hbm.at[j, pl.ds(col, CHUNK)], wsem[b]
            ).wait()

        for b in range(LAG):
            issue_gather(b, b)
        for j in range(NBUF - LAG):
            issue_gather(j + LAG, (j + LAG) % NBUF)
            wait_gather(j, j % NBUF)
            issue_write(j, j % NBUF)

        @pl.loop(NBUF - LAG, nch - LAG, step=NBUF)
        def _grp(g):
            for b in range(NBUF):
                j = g + b
                bj = (NBUF - LAG + b) % NBUF
                wait_write(j + LAG - NBUF, b)
                issue_gather(j + LAG, b)
                wait_gather(j, bj)
                issue_write(j, bj)

        for j in range(nch - LAG, nch):
            wait_gather(j, j % NBUF)
            issue_write(j, j % NBUF)

        for j in range(nch - NBUF, nch):
            wait_write(j, j % NBUF)

    return gather


def kernel(x, word_embed):
    bat, hist = x.shape
    assert bat % (NW * 8) == 0 and bat // NW == CHUNK
    assert (hist - NBUF) % NBUF == 0 and LAG < NBUF
    idx3 = jnp.transpose(x, (1, 0)).reshape(hist, NW, CHUNK)
    idx3 = jnp.transpose(idx3, (1, 0, 2)).astype(jnp.int32)
    out = _make_gather(bat, hist)(word_embed, idx3)
    return jnp.transpose(out, (1, 0, 2))
